# baseline (speedup 1.0000x reference)
import jax
import jax.numpy as jnp
from jax.experimental import pallas as pl
from jax.experimental.pallas import tpu as pltpu

_B, _D, _C = 256, 512, 100000
_UM, _LM = 0.8, 0.45
_UA, _LA = 110.0, 10.0
_LAMBDA_G = 35.0
_SCALE = 64.0

_CB = 1000
_NBLK = _C // _CB
_NEG = -1e30


def _body(xt_ref, xn1_ref, lab_ref, w_ref, o_ref,
          xnt_scr, cm_scr, sm_scr, m_scr, s_scr, t_scr):
    j = pl.program_id(0)

    @pl.when(j == 0)
    def _init():
        xt = xt_ref[...]
        rx = jax.lax.rsqrt(jnp.sum(xt * xt, axis=0, keepdims=True))
        xnt_scr[...] = xt * rx
        a = xn1_ref[...]
        ada = (_UM - _LM) / (_UA - _LA) * (a - _LA) + _LM
        cm_scr[...] = jnp.cos(ada)
        sm_scr[...] = jnp.sin(ada)
        m_scr[...] = jnp.full_like(m_scr, _NEG)
        s_scr[...] = jnp.zeros_like(s_scr)
        t_scr[...] = jnp.zeros_like(t_scr)

    w = w_ref[...]
    rw = jax.lax.rsqrt(jnp.sum(w * w, axis=1, keepdims=True))
    wn = w * rw
    cos = jax.lax.dot_general(
        wn, xnt_scr[...], (((1,), (0,)), ((), ())),
        preferred_element_type=jnp.float32,
        precision=jax.lax.Precision.HIGHEST)
    cos = jnp.clip(cos, -1.0, 1.0)
    sin = jnp.sqrt(1.0 - cos * cos)
    ctm = cos * cm_scr[...] - sin * sm_scr[...]
    ctm = jnp.where(cos > 0.0, ctm, cos)

    rows = j * _CB + jax.lax.broadcasted_iota(jnp.int32, (_CB, _B), 0)
    hit = rows == lab_ref[...]
    logits = _SCALE * jnp.where(hit, ctm, cos)

    bm = jnp.max(logits, axis=0, keepdims=True)
    m_new = jnp.maximum(m_scr[...], bm)
    p = jnp.exp(logits - m_new)
    s_scr[...] = (s_scr[...] * jnp.exp(m_scr[...] - m_new)
                  + jnp.sum(p, axis=0, keepdims=True))
    m_scr[...] = m_new
    t_scr[...] = t_scr[...] + jnp.sum(jnp.where(hit, logits, 0.0),
                                      axis=0, keepdims=True)

    @pl.when(j == _NBLK - 1)
    def _fin():
        lse = jnp.log(s_scr[...]) + m_scr[...]
        a = xn1_ref[...]
        g = a * (1.0 / (_UA * _UA)) + 1.0 / a
        tot = jnp.sum((lse - t_scr[...]) + _LAMBDA_G * g,
                      axis=1, keepdims=True)
        o_ref[...] = tot * (1.0 / _B)


def kernel(x, x_norm, labels, weight):
    xt = x.T
    xn1 = x_norm.reshape(1, _B)
    lab1 = labels.reshape(1, _B).astype(jnp.int32)

    out = pl.pallas_call(
        _body,
        grid=(_NBLK,),
        in_specs=[
            pl.BlockSpec((_D, _B), lambda j: (0, 0)),
            pl.BlockSpec((1, _B), lambda j: (0, 0)),
            pl.BlockSpec((1, _B), lambda j: (0, 0)),
            pl.BlockSpec((_CB, _D), lambda j: (j, 0)),
        ],
        out_specs=pl.BlockSpec((1, 1), lambda j: (0, 0)),
        out_shape=jax.ShapeDtypeStruct((1, 1), jnp.float32),
        scratch_shapes=[
            pltpu.VMEM((_D, _B), jnp.float32),
            pltpu.VMEM((1, _B), jnp.float32),
            pltpu.VMEM((1, _B), jnp.float32),
            pltpu.VMEM((1, _B), jnp.float32),
            pltpu.VMEM((1, _B), jnp.float32),
            pltpu.VMEM((1, _B), jnp.float32),
        ],
        compiler_params=pltpu.CompilerParams(
            dimension_semantics=("arbitrary",),
        ),
        name="magface_loss",
    )(xt, xn1, lab1, weight)
    return out[0, 0]


# speedup vs baseline: 2.9908x; 2.9908x over previous
import jax
import jax.numpy as jnp
from jax.experimental import pallas as pl
from jax.experimental.pallas import tpu as pltpu

_B, _D, _C = 256, 512, 100000
_UM, _LM = 0.8, 0.45
_UA, _LA = 110.0, 10.0
_LAMBDA_G = 35.0
_SCALE = 64.0

_CB = 2000
_NBLK = _C // _CB
_NEG = -1e30


def _body(xt_ref, xn1_ref, lab_ref, w_ref, o_ref,
          xnt_scr, m_scr, s_scr, t_scr):
    j = pl.program_id(0)

    @pl.when(j == 0)
    def _init():
        xt = xt_ref[...]
        rx = jax.lax.rsqrt(jnp.sum(xt * xt, axis=0, keepdims=True))
        xnt_scr[...] = (xt * rx).astype(jnp.bfloat16)
        m_scr[...] = jnp.full_like(m_scr, _NEG)
        s_scr[...] = jnp.zeros_like(s_scr)
        t_scr[...] = jnp.zeros_like(t_scr)

    w = w_ref[...]
    rws = _SCALE * jax.lax.rsqrt(jnp.sum(w * w, axis=1, keepdims=True))
    raw = jax.lax.dot_general(
        w.astype(jnp.bfloat16), xnt_scr[...], (((1,), (0,)), ((), ())),
        preferred_element_type=jnp.float32)
    logits = raw * rws

    rows = j * _CB + jax.lax.broadcasted_iota(jnp.int32, (_CB, _B), 0)
    hit = rows == lab_ref[...]

    bm = jnp.max(logits, axis=0, keepdims=True)
    m_new = jnp.maximum(m_scr[...], bm)
    p = jnp.exp(logits - m_new)
    s_scr[...] = (s_scr[...] * jnp.exp(m_scr[...] - m_new)
                  + jnp.sum(p, axis=0, keepdims=True))
    m_scr[...] = m_new
    t_scr[...] = t_scr[...] + jnp.sum(jnp.where(hit, logits, 0.0),
                                      axis=0, keepdims=True)

    @pl.when(j == _NBLK - 1)
    def _fin():
        m = m_scr[...]
        s0 = t_scr[...]
        cosl = jnp.clip(s0 * (1.0 / _SCALE), -1.0, 1.0)
        a = xn1_ref[...]
        ada = (_UM - _LM) / (_UA - _LA) * (a - _LA) + _LM
        ctm = cosl * jnp.cos(ada) - jnp.sqrt(1.0 - cosl * cosl) * jnp.sin(ada)
        ctm = jnp.where(cosl > 0.0, ctm, cosl)
        sm = _SCALE * ctm
        s_new = (jnp.maximum(s_scr[...] - jnp.exp(s0 - m), 0.0)
                 + jnp.exp(sm - m))
        lse = jnp.log(s_new) + m
        g = a * (1.0 / (_UA * _UA)) + 1.0 / a
        tot = jnp.sum((lse - sm) + _LAMBDA_G * g, axis=1, keepdims=True)
        o_ref[...] = tot * (1.0 / _B)


def kernel(x, x_norm, labels, weight):
    xt = x.T
    xn1 = x_norm.reshape(1, _B)
    lab1 = labels.reshape(1, _B).astype(jnp.int32)

    out = pl.pallas_call(
        _body,
        grid=(_NBLK,),
        in_specs=[
            pl.BlockSpec((_D, _B), lambda j: (0, 0)),
            pl.BlockSpec((1, _B), lambda j: (0, 0)),
            pl.BlockSpec((1, _B), lambda j: (0, 0)),
            pl.BlockSpec((_CB, _D), lambda j: (j, 0)),
        ],
        out_specs=pl.BlockSpec((1, 1), lambda j: (0, 0)),
        out_shape=jax.ShapeDtypeStruct((1, 1), jnp.float32),
        scratch_shapes=[
            pltpu.VMEM((_D, _B), jnp.bfloat16),
            pltpu.VMEM((1, _B), jnp.float32),
            pltpu.VMEM((1, _B), jnp.float32),
            pltpu.VMEM((1, _B), jnp.float32),
        ],
        compiler_params=pltpu.CompilerParams(
            dimension_semantics=("arbitrary",),
            vmem_limit_bytes=56 * 1024 * 1024,
        ),
        name="magface_loss",
    )(xt, xn1, lab1, weight)
    return out[0, 0]
